# Initial kernel scaffold; baseline (speedup 1.0000x reference)
import jax
import jax.numpy as jnp
from jax.experimental import pallas as pl


def kernel(x, router_weight, w_fc, w_proj, ws_fc, ws_proj):
    raise NotImplementedError("write your pallas kernel here")


# speedup vs baseline: 1.8435x; 1.8435x over previous
import functools
import math

import jax
import jax.numpy as jnp
from jax import lax
from jax.experimental import pallas as pl
from jax.experimental.pallas import tpu as pltpu
from jax.experimental.pallas import tpu_sc as plsc

B, T = 2, 2048
N = B * T
D = 1024
E = 8
K = 2
H = 1408
CAP = int(math.ceil(1.25 * K * N / E))
AUXC = 0.01 * 8.0

TRASH = E * CAP
EIN_ROWS = E * CAP + 8

NC, NS, L = 2, 16, 16
NW = NC * NS
TOK_W = N // NW
CH = 32
NCHUNK = TOK_W // CH

ROUTE_BLK = 512
ROUTE_GRID = N // ROUTE_BLK


def _route_body(x_ref, rwt_ref, s0_ref, s1_ref, w0_ref, w1_ref,
                w0b_ref, w1b_ref, aux_ref, carry, psum, cnt):
    i = pl.program_id(0)

    @pl.when(i == 0)
    def _():
        carry[...] = jnp.zeros_like(carry)
        psum[...] = jnp.zeros_like(psum)
        cnt[...] = jnp.zeros_like(cnt)

    xb = x_ref[...]
    logits = jnp.dot(xb, rwt_ref[...], preferred_element_type=jnp.float32)
    m = jnp.max(logits, axis=1, keepdims=True)
    ex = jnp.exp(logits - m)
    probs = ex / jnp.sum(ex, axis=1, keepdims=True)

    lane = lax.broadcasted_iota(jnp.int32, (ROUTE_BLK, E), 1)
    p0 = jnp.max(probs, axis=1, keepdims=True)
    e0 = jnp.min(jnp.where(probs == p0, lane, E), axis=1, keepdims=True)
    oh0 = (lane == e0).astype(jnp.float32)
    probs1 = jnp.where(lane == e0, -1.0, probs)
    p1 = jnp.max(probs1, axis=1, keepdims=True)
    e1 = jnp.min(jnp.where(probs1 == p1, lane, E), axis=1, keepdims=True)
    oh1 = (lane == e1).astype(jnp.float32)

    denom = p0 + p1 + 1e-9
    w0 = p0 / denom
    w1 = p1 / denom

    S = oh0 + oh1
    r = lax.broadcasted_iota(jnp.int32, (ROUTE_BLK, ROUTE_BLK), 0)
    c = lax.broadcasted_iota(jnp.int32, (ROUTE_BLK, ROUTE_BLK), 1)
    tri = (c < r).astype(jnp.float32)
    ec = jnp.dot(tri, S, preferred_element_type=jnp.float32) + carry[...]
    pos0 = jnp.sum(ec * oh0, axis=1, keepdims=True)
    pos1 = jnp.sum(ec * oh1, axis=1, keepdims=True)
    keep0 = (pos0 < CAP).astype(jnp.float32)
    keep1 = (pos1 < CAP).astype(jnp.float32)
    slot0 = e0 * CAP + jnp.minimum(pos0, CAP - 1).astype(jnp.int32)
    slot1 = e1 * CAP + jnp.minimum(pos1, CAP - 1).astype(jnp.int32)

    w0k = w0 * keep0
    w1k = w1 * keep1
    s0_ref[...] = slot0.reshape(1, ROUTE_BLK, 1)
    s1_ref[...] = slot1.reshape(1, ROUTE_BLK, 1)
    w0_ref[...] = w0k.reshape(1, ROUTE_BLK, 1)
    w1_ref[...] = w1k.reshape(1, ROUTE_BLK, 1)
    w0b_ref[...] = jnp.broadcast_to(w0k, (ROUTE_BLK, L)).reshape(1, ROUTE_BLK, L)
    w1b_ref[...] = jnp.broadcast_to(w1k, (ROUTE_BLK, L)).reshape(1, ROUTE_BLK, L)

    carry[...] = carry[...] + jnp.sum(S, axis=0, keepdims=True)
    psum[...] = psum[...] + jnp.sum(probs, axis=0, keepdims=True)
    cnt[...] = cnt[...] + jnp.sum(oh0, axis=0, keepdims=True)

    @pl.when(i == ROUTE_GRID - 1)
    def _():
        f = cnt[...] / jnp.float32(N)
        p = psum[...] / jnp.float32(N)
        aux_ref[...] = (jnp.sum(f * p) * jnp.float32(AUXC)).reshape(1, 1)


def _route(x2d, rwt):
    return pl.pallas_call(
        _route_body,
        grid=(ROUTE_GRID,),
        in_specs=[
            pl.BlockSpec((ROUTE_BLK, D), lambda i: (i, 0)),
            pl.BlockSpec((D, E), lambda i: (0, 0)),
        ],
        out_specs=[
            pl.BlockSpec((1, ROUTE_BLK, 1), lambda i: (i, 0, 0)),
            pl.BlockSpec((1, ROUTE_BLK, 1), lambda i: (i, 0, 0)),
            pl.BlockSpec((1, ROUTE_BLK, 1), lambda i: (i, 0, 0)),
            pl.BlockSpec((1, ROUTE_BLK, 1), lambda i: (i, 0, 0)),
            pl.BlockSpec((1, ROUTE_BLK, L), lambda i: (i, 0, 0)),
            pl.BlockSpec((1, ROUTE_BLK, L), lambda i: (i, 0, 0)),
            pl.BlockSpec((1, 1), lambda i: (0, 0)),
        ],
        out_shape=[
            jax.ShapeDtypeStruct((ROUTE_GRID, ROUTE_BLK, 1), jnp.int32),
            jax.ShapeDtypeStruct((ROUTE_GRID, ROUTE_BLK, 1), jnp.int32),
            jax.ShapeDtypeStruct((ROUTE_GRID, ROUTE_BLK, 1), jnp.float32),
            jax.ShapeDtypeStruct((ROUTE_GRID, ROUTE_BLK, 1), jnp.float32),
            jax.ShapeDtypeStruct((ROUTE_GRID, ROUTE_BLK, L), jnp.float32),
            jax.ShapeDtypeStruct((ROUTE_GRID, ROUTE_BLK, L), jnp.float32),
            jax.ShapeDtypeStruct((1, 1), jnp.float32),
        ],
        scratch_shapes=[
            pltpu.VMEM((1, E), jnp.float32),
            pltpu.VMEM((1, E), jnp.float32),
            pltpu.VMEM((1, E), jnp.float32),
        ],
    )(x2d, rwt)


@functools.lru_cache(maxsize=None)
def _sc_mesh():
    return plsc.VectorSubcoreMesh(core_axis_name="c", subcore_axis_name="s",
                                  num_cores=NC, num_subcores=NS)


def _dispatch_body(x_hbm, s0_hbm, s1_hbm, w0_hbm, w1_hbm, ein_hbm,
                   xbuf, s0b, s1b, w0b, w1b, sc0, sc1):
    wid = lax.axis_index("s") * NC + lax.axis_index("c")
    for ci in range(NCHUNK):
        base = wid * TOK_W + ci * CH
        pltpu.sync_copy(x_hbm.at[pl.ds(base, CH)], xbuf)
        pltpu.sync_copy(s0_hbm.at[pl.ds(base, CH)], s0b)
        pltpu.sync_copy(s1_hbm.at[pl.ds(base, CH)], s1b)
        pltpu.sync_copy(w0_hbm.at[pl.ds(base, CH)], w0b)
        pltpu.sync_copy(w1_hbm.at[pl.ds(base, CH)], w1b)
        for j in range(CH // L):
            sl = pl.ds(j * L, L)
            sc0[sl] = jnp.where(w0b[sl] > 0.0, s0b[sl], TRASH)
            sc1[sl] = jnp.where(w1b[sl] > 0.0, s1b[sl], TRASH)
        pltpu.sync_copy(xbuf, ein_hbm.at[sc0])
        pltpu.sync_copy(xbuf, ein_hbm.at[sc1])


@functools.lru_cache(maxsize=None)
def _dispatch_kernel():
    return pl.kernel(
        _dispatch_body,
        out_type=jax.ShapeDtypeStruct((EIN_ROWS, D), jnp.float32),
        mesh=_sc_mesh(),
        scratch_types=[
            pltpu.VMEM((CH, D), jnp.float32),
            pltpu.VMEM((CH,), jnp.int32),
            pltpu.VMEM((CH,), jnp.int32),
            pltpu.VMEM((CH,), jnp.float32),
            pltpu.VMEM((CH,), jnp.float32),
            pltpu.VMEM((CH,), jnp.int32),
            pltpu.VMEM((CH,), jnp.int32),
        ],
    )


FFN_BLK = 256


def _ffn_body(xin_ref, wfc_ref, wpj_ref, out_ref):
    h = jnp.dot(xin_ref[...], wfc_ref[0], preferred_element_type=jnp.float32)
    a = jnp.square(jnp.maximum(h, 0.0))
    out_ref[...] = jnp.dot(a, wpj_ref[0], preferred_element_type=jnp.float32)


def _routed_ffn(ein, w_fc, w_proj):
    rblk = CAP // FFN_BLK
    return pl.pallas_call(
        _ffn_body,
        grid=(E, rblk),
        in_specs=[
            pl.BlockSpec((FFN_BLK, D), lambda e, r: (e * (CAP // FFN_BLK) + r, 0)),
            pl.BlockSpec((1, D, H), lambda e, r: (e, 0, 0)),
            pl.BlockSpec((1, H, D), lambda e, r: (e, 0, 0)),
        ],
        out_specs=pl.BlockSpec((FFN_BLK, D), lambda e, r: (e * (CAP // FFN_BLK) + r, 0)),
        out_shape=jax.ShapeDtypeStruct((E * CAP, D), jnp.float32),
    )(ein, w_fc, w_proj)


def _shared_ffn(x2d, ws_fc, ws_proj):
    return pl.pallas_call(
        _ffn_body,
        grid=(N // FFN_BLK,),
        in_specs=[
            pl.BlockSpec((FFN_BLK, D), lambda i: (i, 0)),
            pl.BlockSpec((1, D, H), lambda i: (0, 0, 0)),
            pl.BlockSpec((1, H, D), lambda i: (0, 0, 0)),
        ],
        out_specs=pl.BlockSpec((FFN_BLK, D), lambda i: (i, 0)),
        out_shape=jax.ShapeDtypeStruct((N, D), jnp.float32),
    )(x2d, ws_fc, ws_proj)


def _combine_body(eo_hbm, sh_hbm, s0_hbm, s1_hbm, w0m_hbm, w1m_hbm, y_hbm,
                  r0, r1, shb, s0b, s1b, w0b, w1b, sem):
    wid = lax.axis_index("s") * NC + lax.axis_index("c")
    for ci in range(NCHUNK):
        base = wid * TOK_W + ci * CH
        pltpu.sync_copy(s0_hbm.at[pl.ds(base, CH)], s0b)
        pltpu.sync_copy(s1_hbm.at[pl.ds(base, CH)], s1b)
        pltpu.sync_copy(w0m_hbm.at[pl.ds(base, CH)], w0b)
        pltpu.sync_copy(w1m_hbm.at[pl.ds(base, CH)], w1b)
        pltpu.sync_copy(sh_hbm.at[pl.ds(base, CH)], shb)
        pltpu.async_copy(eo_hbm.at[s0b], r0, sem).wait()
        pltpu.async_copy(eo_hbm.at[s1b], r1, sem).wait()

        def row_fn(row, carry):
            w0v = w0b[row, :]
            w1v = w1b[row, :]
            for d in range(0, D, L):
                sl = pl.ds(d, L)
                shb[row, sl] = shb[row, sl] + w0v * r0[row, sl] + w1v * r1[row, sl]
            return carry

        lax.fori_loop(0, CH, row_fn, 0)
        pltpu.sync_copy(shb, y_hbm.at[pl.ds(base, CH)])


@functools.lru_cache(maxsize=None)
def _combine_kernel():
    return pl.kernel(
        _combine_body,
        out_type=jax.ShapeDtypeStruct((N, D), jnp.float32),
        mesh=_sc_mesh(),
        scratch_types=[
            pltpu.VMEM((CH, D), jnp.float32),
            pltpu.VMEM((CH, D), jnp.float32),
            pltpu.VMEM((CH, D), jnp.float32),
            pltpu.VMEM((CH,), jnp.int32),
            pltpu.VMEM((CH,), jnp.int32),
            pltpu.VMEM((CH, L), jnp.float32),
            pltpu.VMEM((CH, L), jnp.float32),
            pltpu.SemaphoreType.DMA,
        ],
    )


def kernel(x, router_weight, w_fc, w_proj, ws_fc, ws_proj):
    x2d = x.reshape(N, D)
    rwt = router_weight.T

    s0_o, s1_o, w0_o, w1_o, w0b_o, w1b_o, aux_o = _route(x2d, rwt)
    s0 = s0_o.reshape(N)
    s1 = s1_o.reshape(N)
    w0 = w0_o.reshape(N)
    w1 = w1_o.reshape(N)
    w0m = w0b_o.reshape(N, L)
    w1m = w1b_o.reshape(N, L)

    ein = _dispatch_kernel()(x2d, s0, s1, w0, w1)
    eo = _routed_ffn(ein, w_fc, w_proj)
    sh = _shared_ffn(x2d, ws_fc, ws_proj)
    y = _combine_kernel()(eo, sh, s0, s1, w0m, w1m)

    return y.reshape(B, T, D), aux_o.reshape(())
